# baseline (speedup 1.0000x reference)
import functools

import jax
import jax.numpy as jnp
from jax import lax
from jax.experimental import pallas as pl
from jax.experimental.pallas import tpu as pltpu
from jax.experimental.pallas import tpu_sc as plsc

N_NODES = 10000
F = 128
N_EDGES = 320000

NC = 2
NS = 16
NW = NC * NS

NPAD = 10240
ROWS_PER_SUB = NPAD // NS
FH = F // NC

SLOTS_PER_S = 20480
EPAD = NS * SLOTS_PER_S
CHUNK = 128
NBUF = 8
STAGES = 4
HCH = SLOTS_PER_S // CHUNK // STAGES

SLOTS_PER_W = EPAD // NW
DCHUNK = 128
DCH = SLOTS_PER_W // DCHUNK

_mesh = plsc.VectorSubcoreMesh(core_axis_name="c", subcore_axis_name="s")


@functools.partial(
    pl.kernel,
    mesh=_mesh,
    out_type=jax.ShapeDtypeStruct((NC, NPAD, 16), jnp.float32),
    scratch_types=[
        pltpu.VMEM((DCH, DCHUNK), jnp.int32),
        pltpu.VMEM((DCHUNK, 16), jnp.float32),
        pltpu.VMEM((16, 16), jnp.float32),
        pltpu.VMEM_SHARED((NPAD, 16), jnp.float32),
    ],
)
def _sc_degree(dst_hbm, out_hbm, dstv, ones, zb, acc):
    c = lax.axis_index("c")
    s = lax.axis_index("s")
    wid = c * NS + s

    @pl.loop(0, DCHUNK)
    def _(r):
        ones[r, pl.ds(0, 16)] = jnp.ones((16,), jnp.float32)

    @pl.loop(0, 16)
    def _(r):
        zb[r, pl.ds(0, 16)] = jnp.zeros((16,), jnp.float32)

    @pl.loop(0, ROWS_PER_SUB // 16)
    def _(t):
        pltpu.sync_copy(zb, acc.at[pl.ds(s * ROWS_PER_SUB + t * 16, 16)])

    pltpu.sync_copy(dst_hbm.at[wid], dstv)
    plsc.subcore_barrier()

    @pl.loop(0, DCH)
    def _(j):
        pltpu.sync_copy(ones, acc.at[dstv.at[j]], add=True)

    plsc.subcore_barrier()
    pltpu.sync_copy(
        acc.at[pl.ds(s * ROWS_PER_SUB, ROWS_PER_SUB)],
        out_hbm.at[c, pl.ds(s * ROWS_PER_SUB, ROWS_PER_SUB)],
    )


@functools.partial(
    pl.kernel,
    mesh=_mesh,
    out_type=jax.ShapeDtypeStruct((NC, NPAD, FH), jnp.float32),
    scratch_types=[
        pltpu.VMEM((HCH, CHUNK), jnp.int32),
        pltpu.VMEM((HCH, CHUNK), jnp.int32),
        pltpu.VMEM((NBUF, CHUNK, FH), jnp.float32),
        pltpu.VMEM_SHARED((NPAD, FH), jnp.float32),
        pltpu.SemaphoreType.DMA,
        pltpu.SemaphoreType.DMA,
        pltpu.SemaphoreType.DMA,
        pltpu.SemaphoreType.DMA,
        pltpu.SemaphoreType.DMA,
        pltpu.SemaphoreType.DMA,
        pltpu.SemaphoreType.DMA,
        pltpu.SemaphoreType.DMA,
        pltpu.SemaphoreType.DMA,
        pltpu.SemaphoreType.DMA,
    ],
    compiler_params=pltpu.CompilerParams(use_tc_tiling_on_sc=False),
)
def _sc_propagate(hp_hbm, src_hbm, dst_hbm, out_hbm, srcv, dstv, rows, acc,
                  g0, g1, g2, g3, g4, g5, g6, g7, sa, sb):
    c = lax.axis_index("c")
    s = lax.axis_index("s")
    hpc = hp_hbm.at[c]

    @pl.loop(0, CHUNK)
    def _(r):
        @pl.loop(0, FH // 16)
        def _(g):
            rows[0, r, pl.ds(g * 16, 16)] = jnp.zeros((16,), jnp.float32)

    @pl.loop(0, ROWS_PER_SUB // CHUNK)
    def _(t):
        pltpu.sync_copy(rows.at[0],
                        acc.at[pl.ds(s * ROWS_PER_SUB + t * CHUNK, CHUNK)])

    plsc.subcore_barrier()

    gsems = (g0, g1, g2, g3, g4, g5, g6, g7)
    for h in range(STAGES):
        pltpu.sync_copy(src_hbm.at[s, h], srcv)
        pltpu.sync_copy(dst_hbm.at[s, h], dstv)

        @pl.loop(0, HCH, step=NBUF)
        def _(t):
            g = [
                pltpu.async_copy(hpc.at[srcv.at[t + b]], rows.at[b], gsems[b])
                for b in range(NBUF)
            ]
            sca = []
            for b in range(NBUF // 2):
                g[b].wait()
                sca.append(pltpu.async_copy(
                    rows.at[b], acc.at[dstv.at[t + b]], sa, add=True))
            scb = []
            for b in range(NBUF // 2, NBUF):
                g[b].wait()
                scb.append(pltpu.async_copy(
                    rows.at[b], acc.at[dstv.at[t + b]], sb, add=True))
            for d in sca:
                d.wait()
            for d in scb:
                d.wait()

    plsc.subcore_barrier()
    pltpu.sync_copy(
        acc.at[pl.ds(s * ROWS_PER_SUB, ROWS_PER_SUB)],
        out_hbm.at[c, pl.ds(s * ROWS_PER_SUB, ROWS_PER_SUB)],
    )


_BR = 1024


def _dinv_block(degp):
    deg = degp[0, :, 0] + degp[1, :, 0] + 1.0
    return lax.rsqrt(deg)[:, None]


def _tc_first(x_ref, w_ref, degp_ref, o_ref):
    dinv = _dinv_block(degp_ref[...])
    h = jnp.dot(x_ref[...], w_ref[...], preferred_element_type=jnp.float32)
    h = h * dinv
    o_ref[0, :, :] = h[:, :FH]
    o_ref[1, :, :] = h[:, FH:]


def _tc_mid(p_ref, hp_ref, degp_ref, b_ref, w_ref, o_ref):
    dinv = _dinv_block(degp_ref[...])
    agg = jnp.concatenate([p_ref[0] + hp_ref[0], p_ref[1] + hp_ref[1]], axis=1)
    t = dinv * agg + b_ref[...]
    a = jnp.where(t >= 0.0, t, 0.2 * t)
    h = jnp.dot(a, w_ref[...], preferred_element_type=jnp.float32)
    h = h * dinv
    o_ref[0, :, :] = h[:, :FH]
    o_ref[1, :, :] = h[:, FH:]


def _tc_last(p_ref, hp_ref, degp_ref, b_ref, o_ref):
    dinv = _dinv_block(degp_ref[...])
    agg = jnp.concatenate([p_ref[0] + hp_ref[0], p_ref[1] + hp_ref[1]], axis=1)
    t = dinv * agg + b_ref[...]
    m = jnp.max(t, axis=1, keepdims=True)
    e = jnp.exp(t - m)
    lse = jnp.log(jnp.sum(e, axis=1, keepdims=True))
    o_ref[...] = (t - m) - lse


_row_spec = pl.BlockSpec((_BR, F), lambda i: (i, 0))
_half_spec = pl.BlockSpec((NC, _BR, FH), lambda i: (0, i, 0))
_degp_spec = pl.BlockSpec((NC, _BR, 16), lambda i: (0, i, 0))
_w_spec = pl.BlockSpec((F, F), lambda i: (0, 0))
_b_spec = pl.BlockSpec((1, F), lambda i: (0, 0))
_grid = (NPAD // _BR,)
_out_rows = jax.ShapeDtypeStruct((NPAD, F), jnp.float32)
_out_halves = jax.ShapeDtypeStruct((NC, NPAD, FH), jnp.float32)


def kernel(x, edge_index, W1, b1, W2, b2):
    src = edge_index[0].astype(jnp.int32)
    dst = edge_index[1].astype(jnp.int32)

    e_per_s = N_EDGES // NS
    pad_per_s = SLOTS_PER_S - e_per_s
    padv = (N_NODES + (jnp.arange(NS * pad_per_s, dtype=jnp.int32)
                       % (NPAD - N_NODES))).reshape(NS, pad_per_s)
    src4 = jnp.concatenate([src.reshape(NS, e_per_s), padv],
                           axis=1).reshape(NS, STAGES, HCH, CHUNK)
    dst4 = jnp.concatenate([dst.reshape(NS, e_per_s), padv],
                           axis=1).reshape(NS, STAGES, HCH, CHUNK)

    e_per_w = N_EDGES // NW
    pad_per_w = SLOTS_PER_W - e_per_w
    padw = (N_NODES + (jnp.arange(NW * pad_per_w, dtype=jnp.int32)
                       % (NPAD - N_NODES))).reshape(NW, pad_per_w)
    dstd = jnp.concatenate([dst.reshape(NW, e_per_w), padw],
                           axis=1).reshape(NW, DCH, DCHUNK)

    xpad = jnp.zeros((NPAD, F), jnp.float32).at[:N_NODES].set(x)
    b1r = b1.reshape(1, F)
    b2r = b2.reshape(1, F)

    degp = _sc_degree(dstd)

    h1p = pl.pallas_call(
        _tc_first,
        grid=_grid,
        in_specs=[_row_spec, _w_spec, _degp_spec],
        out_specs=_half_spec,
        out_shape=_out_halves,
    )(xpad, W1, degp)

    p1 = _sc_propagate(h1p, src4, dst4)

    h2p = pl.pallas_call(
        _tc_mid,
        grid=_grid,
        in_specs=[_half_spec, _half_spec, _degp_spec, _b_spec, _w_spec],
        out_specs=_half_spec,
        out_shape=_out_halves,
    )(p1, h1p, degp, b1r, W2)

    p2 = _sc_propagate(h2p, src4, dst4)

    out = pl.pallas_call(
        _tc_last,
        grid=_grid,
        in_specs=[_half_spec, _half_spec, _degp_spec, _b_spec],
        out_specs=_row_spec,
        out_shape=_out_rows,
    )(p2, h2p, degp, b2r)

    return out[:N_NODES]


# speedup vs baseline: 1.1871x; 1.1871x over previous
import functools

import jax
import jax.numpy as jnp
from jax import lax
from jax.experimental import pallas as pl
from jax.experimental.pallas import tpu as pltpu
from jax.experimental.pallas import tpu_sc as plsc

N_NODES = 10000
F = 128
N_EDGES = 320000

NC = 2
NS = 16
NW = NC * NS

NPAD = 10240
ROWS_PER_SUB = NPAD // NS

SLOTS_PER_W = 10240
EPAD = NW * SLOTS_PER_W

CHUNK = 64
NBUF = 4
STAGES = 4
HCH = SLOTS_PER_W // CHUNK // STAGES

DCHUNK = 128
DCH = SLOTS_PER_W // DCHUNK

_mesh = plsc.VectorSubcoreMesh(core_axis_name="c", subcore_axis_name="s")


@functools.partial(
    pl.kernel,
    mesh=_mesh,
    out_type=jax.ShapeDtypeStruct((NC, NPAD, 16), jnp.float32),
    scratch_types=[
        pltpu.VMEM((DCH, DCHUNK), jnp.int32),
        pltpu.VMEM((DCHUNK, 16), jnp.float32),
        pltpu.VMEM((16, 16), jnp.float32),
        pltpu.VMEM_SHARED((NPAD, 16), jnp.float32),
    ],
)
def _sc_degree(dst_hbm, out_hbm, dstv, ones, zb, acc):
    c = lax.axis_index("c")
    s = lax.axis_index("s")
    wid = c * NS + s

    @pl.loop(0, DCHUNK)
    def _(r):
        ones[r, pl.ds(0, 16)] = jnp.ones((16,), jnp.float32)

    @pl.loop(0, 16)
    def _(r):
        zb[r, pl.ds(0, 16)] = jnp.zeros((16,), jnp.float32)

    @pl.loop(0, ROWS_PER_SUB // 16)
    def _(t):
        pltpu.sync_copy(zb, acc.at[pl.ds(s * ROWS_PER_SUB + t * 16, 16)])

    pltpu.sync_copy(dst_hbm.at[wid], dstv)
    plsc.subcore_barrier()

    @pl.loop(0, DCH)
    def _(j):
        pltpu.sync_copy(ones, acc.at[dstv.at[j]], add=True)

    plsc.subcore_barrier()
    pltpu.sync_copy(
        acc.at[pl.ds(s * ROWS_PER_SUB, ROWS_PER_SUB)],
        out_hbm.at[c, pl.ds(s * ROWS_PER_SUB, ROWS_PER_SUB)],
    )


@functools.partial(
    pl.kernel,
    mesh=_mesh,
    out_type=jax.ShapeDtypeStruct((NC, NPAD, F), jnp.float32),
    scratch_types=[
        pltpu.VMEM((HCH, CHUNK), jnp.int32),
        pltpu.VMEM((HCH, CHUNK), jnp.int32),
        pltpu.VMEM((NBUF, CHUNK, F), jnp.float32),
        pltpu.VMEM_SHARED((NPAD, F), jnp.float32),
        pltpu.SemaphoreType.DMA,
        pltpu.SemaphoreType.DMA,
        pltpu.SemaphoreType.DMA,
        pltpu.SemaphoreType.DMA,
        pltpu.SemaphoreType.DMA,
        pltpu.SemaphoreType.DMA,
        pltpu.SemaphoreType.DMA,
        pltpu.SemaphoreType.DMA,
    ],
)
def _sc_propagate(hp_hbm, src_hbm, dst_hbm, out_hbm, srcv, dstv, rows, acc,
                  g0, g1, g2, g3, s0, s1, s2, s3):
    c = lax.axis_index("c")
    s = lax.axis_index("s")
    wid = c * NS + s

    @pl.loop(0, CHUNK)
    def _(r):
        @pl.loop(0, F // 16)
        def _(g):
            rows[0, r, pl.ds(g * 16, 16)] = jnp.zeros((16,), jnp.float32)

    @pl.loop(0, ROWS_PER_SUB // CHUNK)
    def _(t):
        pltpu.sync_copy(rows.at[0],
                        acc.at[pl.ds(s * ROWS_PER_SUB + t * CHUNK, CHUNK)])

    plsc.subcore_barrier()

    gsems = (g0, g1, g2, g3)
    ssems = (s0, s1, s2, s3)
    for h in range(STAGES):
        pltpu.sync_copy(src_hbm.at[wid, h], srcv)
        pltpu.sync_copy(dst_hbm.at[wid, h], dstv)

        @pl.loop(0, HCH, step=NBUF)
        def _(t):
            g = []
            for b in range(NBUF):
                @pl.when(t > 0)
                def _():
                    pltpu.make_async_copy(
                        rows.at[b], acc.at[dstv.at[t - NBUF + b]],
                        ssems[b]).wait()
                g.append(pltpu.async_copy(
                    hp_hbm.at[srcv.at[t + b]], rows.at[b], gsems[b]))
            for b in range(NBUF):
                g[b].wait()
                pltpu.async_copy(
                    rows.at[b], acc.at[dstv.at[t + b]], ssems[b], add=True)

        for b in range(NBUF):
            pltpu.make_async_copy(
                rows.at[b], acc.at[dstv.at[HCH - NBUF + b]], ssems[b]).wait()

    plsc.subcore_barrier()
    pltpu.sync_copy(
        acc.at[pl.ds(s * ROWS_PER_SUB, ROWS_PER_SUB)],
        out_hbm.at[c, pl.ds(s * ROWS_PER_SUB, ROWS_PER_SUB)],
    )


_BR = 1024


def _dinv_block(degp):
    deg = degp[0, :, 0] + degp[1, :, 0] + 1.0
    return lax.rsqrt(deg)[:, None]


def _tc_first(x_ref, w_ref, degp_ref, o_ref):
    dinv = _dinv_block(degp_ref[...])
    h = jnp.dot(x_ref[...], w_ref[...], preferred_element_type=jnp.float32)
    o_ref[...] = h * dinv


def _tc_mid(p_ref, hp_ref, degp_ref, b_ref, w_ref, o_ref):
    dinv = _dinv_block(degp_ref[...])
    t = dinv * (p_ref[0] + p_ref[1] + hp_ref[...]) + b_ref[...]
    a = jnp.where(t >= 0.0, t, 0.2 * t)
    h = jnp.dot(a, w_ref[...], preferred_element_type=jnp.float32)
    o_ref[...] = h * dinv


def _tc_last(p_ref, hp_ref, degp_ref, b_ref, o_ref):
    dinv = _dinv_block(degp_ref[...])
    t = dinv * (p_ref[0] + p_ref[1] + hp_ref[...]) + b_ref[...]
    m = jnp.max(t, axis=1, keepdims=True)
    e = jnp.exp(t - m)
    lse = jnp.log(jnp.sum(e, axis=1, keepdims=True))
    o_ref[...] = (t - m) - lse


_row_spec = pl.BlockSpec((_BR, F), lambda i: (i, 0))
_part_spec = pl.BlockSpec((NC, _BR, F), lambda i: (0, i, 0))
_degp_spec = pl.BlockSpec((NC, _BR, 16), lambda i: (0, i, 0))
_w_spec = pl.BlockSpec((F, F), lambda i: (0, 0))
_b_spec = pl.BlockSpec((1, F), lambda i: (0, 0))
_grid = (NPAD // _BR,)
_out_rows = jax.ShapeDtypeStruct((NPAD, F), jnp.float32)


def kernel(x, edge_index, W1, b1, W2, b2):
    src = edge_index[0].astype(jnp.int32)
    dst = edge_index[1].astype(jnp.int32)
    e_per_w = N_EDGES // NW
    pad_per_w = SLOTS_PER_W - e_per_w
    padv = (N_NODES + (jnp.arange(NW * pad_per_w, dtype=jnp.int32)
                       % (NPAD - N_NODES))).reshape(NW, pad_per_w)
    srcf = jnp.concatenate([src.reshape(NW, e_per_w), padv], axis=1)
    dstf = jnp.concatenate([dst.reshape(NW, e_per_w), padv], axis=1)
    src3 = srcf.reshape(NW, STAGES, HCH, CHUNK)
    dst3 = dstf.reshape(NW, STAGES, HCH, CHUNK)
    dstd = dstf.reshape(NW, DCH, DCHUNK)

    xpad = jnp.zeros((NPAD, F), jnp.float32).at[:N_NODES].set(x)
    b1r = b1.reshape(1, F)
    b2r = b2.reshape(1, F)

    degp = _sc_degree(dstd)

    h1p = pl.pallas_call(
        _tc_first,
        grid=_grid,
        in_specs=[_row_spec, _w_spec, _degp_spec],
        out_specs=_row_spec,
        out_shape=_out_rows,
    )(xpad, W1, degp)

    p1 = _sc_propagate(h1p, src3, dst3)

    h2p = pl.pallas_call(
        _tc_mid,
        grid=_grid,
        in_specs=[_part_spec, _row_spec, _degp_spec, _b_spec, _w_spec],
        out_specs=_row_spec,
        out_shape=_out_rows,
    )(p1, h1p, degp, b1r, W2)

    p2 = _sc_propagate(h2p, src3, dst3)

    out = pl.pallas_call(
        _tc_last,
        grid=_grid,
        in_specs=[_part_spec, _row_spec, _degp_spec, _b_spec],
        out_specs=_row_spec,
        out_shape=_out_rows,
    )(p2, h2p, degp, b2r)

    return out[:N_NODES]


# speedup vs baseline: 1.2017x; 1.0123x over previous
import functools

import jax
import jax.numpy as jnp
from jax import lax
from jax.experimental import pallas as pl
from jax.experimental.pallas import tpu as pltpu
from jax.experimental.pallas import tpu_sc as plsc

N_NODES = 10000
F = 128
N_EDGES = 320000

NC = 2
NS = 16
NW = NC * NS

NPAD = 10240
ROWS_PER_SUB = NPAD // NS

SLOTS_PER_W = 10240
EPAD = NW * SLOTS_PER_W

CHUNK = 64
NBUF = 4
STAGES = 4
HCH = SLOTS_PER_W // CHUNK // STAGES

DCHUNK = 128
DCH = SLOTS_PER_W // DCHUNK

_mesh = plsc.VectorSubcoreMesh(core_axis_name="c", subcore_axis_name="s")


@functools.partial(
    pl.kernel,
    mesh=_mesh,
    out_type=jax.ShapeDtypeStruct((NC, NPAD, 16), jnp.float32),
    scratch_types=[
        pltpu.VMEM((DCH, DCHUNK), jnp.int32),
        pltpu.VMEM((DCHUNK, 16), jnp.float32),
        pltpu.VMEM((16, 16), jnp.float32),
        pltpu.VMEM_SHARED((NPAD, 16), jnp.float32),
        pltpu.SemaphoreType.DMA,
    ],
)
def _sc_degree(dst_hbm, out_hbm, dstv, ones, zb, acc, dsem):
    c = lax.axis_index("c")
    s = lax.axis_index("s")
    wid = c * NS + s

    @pl.loop(0, DCHUNK)
    def _(r):
        ones[r, pl.ds(0, 16)] = jnp.ones((16,), jnp.float32)

    @pl.loop(0, 16)
    def _(r):
        zb[r, pl.ds(0, 16)] = jnp.zeros((16,), jnp.float32)

    @pl.loop(0, ROWS_PER_SUB // 16)
    def _(t):
        pltpu.sync_copy(zb, acc.at[pl.ds(s * ROWS_PER_SUB + t * 16, 16)])

    pltpu.sync_copy(dst_hbm.at[wid], dstv)
    plsc.subcore_barrier()

    @pl.loop(0, DCH, step=8)
    def _(j):
        @pl.when(j > 0)
        def _():
            for b in range(8):
                pltpu.make_async_copy(
                    ones, acc.at[dstv.at[j - 8 + b]], dsem).wait()
        for b in range(8):
            pltpu.async_copy(ones, acc.at[dstv.at[j + b]], dsem, add=True)

    for b in range(8):
        pltpu.make_async_copy(ones, acc.at[dstv.at[DCH - 8 + b]], dsem).wait()

    plsc.subcore_barrier()
    pltpu.sync_copy(
        acc.at[pl.ds(s * ROWS_PER_SUB, ROWS_PER_SUB)],
        out_hbm.at[c, pl.ds(s * ROWS_PER_SUB, ROWS_PER_SUB)],
    )


@functools.partial(
    pl.kernel,
    mesh=_mesh,
    out_type=jax.ShapeDtypeStruct((NC, NPAD, F), jnp.float32),
    scratch_types=[
        pltpu.VMEM((HCH, CHUNK), jnp.int32),
        pltpu.VMEM((HCH, CHUNK), jnp.int32),
        pltpu.VMEM((NBUF, CHUNK, F), jnp.float32),
        pltpu.VMEM_SHARED((NPAD, F), jnp.float32),
        pltpu.SemaphoreType.DMA,
        pltpu.SemaphoreType.DMA,
        pltpu.SemaphoreType.DMA,
        pltpu.SemaphoreType.DMA,
        pltpu.SemaphoreType.DMA,
        pltpu.SemaphoreType.DMA,
        pltpu.SemaphoreType.DMA,
        pltpu.SemaphoreType.DMA,
    ],
)
def _sc_propagate(hp_hbm, src_hbm, dst_hbm, out_hbm, srcv, dstv, rows, acc,
                  g0, g1, g2, g3, s0, s1, s2, s3):
    c = lax.axis_index("c")
    s = lax.axis_index("s")
    wid = c * NS + s

    @pl.loop(0, CHUNK)
    def _(r):
        @pl.loop(0, F // 16)
        def _(g):
            rows[0, r, pl.ds(g * 16, 16)] = jnp.zeros((16,), jnp.float32)

    @pl.loop(0, ROWS_PER_SUB // CHUNK)
    def _(t):
        pltpu.sync_copy(rows.at[0],
                        acc.at[pl.ds(s * ROWS_PER_SUB + t * CHUNK, CHUNK)])

    plsc.subcore_barrier()

    gsems = (g0, g1, g2, g3)
    ssems = (s0, s1, s2, s3)
    for h in range(STAGES):
        pltpu.sync_copy(src_hbm.at[wid, h], srcv)
        pltpu.sync_copy(dst_hbm.at[wid, h], dstv)

        @pl.loop(0, HCH, step=NBUF)
        def _(t):
            g = []
            for b in range(NBUF):
                @pl.when(t > 0)
                def _():
                    pltpu.make_async_copy(
                        rows.at[b], acc.at[dstv.at[t - NBUF + b]],
                        ssems[b]).wait()
                g.append(pltpu.async_copy(
                    hp_hbm.at[srcv.at[t + b]], rows.at[b], gsems[b]))
            for b in range(NBUF):
                g[b].wait()
                pltpu.async_copy(
                    rows.at[b], acc.at[dstv.at[t + b]], ssems[b], add=True)

        for b in range(NBUF):
            pltpu.make_async_copy(
                rows.at[b], acc.at[dstv.at[HCH - NBUF + b]], ssems[b]).wait()

    plsc.subcore_barrier()
    pltpu.sync_copy(
        acc.at[pl.ds(s * ROWS_PER_SUB, ROWS_PER_SUB)],
        out_hbm.at[c, pl.ds(s * ROWS_PER_SUB, ROWS_PER_SUB)],
    )


_BR = 1024


def _dinv_block(degp):
    deg = degp[0, :, 0] + degp[1, :, 0] + 1.0
    return lax.rsqrt(deg)[:, None]


def _tc_matmul(x_ref, w_ref, o_ref):
    o_ref[...] = jnp.dot(x_ref[...], w_ref[...],
                         preferred_element_type=jnp.float32)


def _tc_scale(h_ref, degp_ref, o_ref):
    dinv = _dinv_block(degp_ref[...])
    o_ref[...] = h_ref[...] * dinv


def _tc_mid(p_ref, hp_ref, degp_ref, b_ref, w_ref, o_ref):
    dinv = _dinv_block(degp_ref[...])
    t = dinv * (p_ref[0] + p_ref[1] + hp_ref[...]) + b_ref[...]
    a = jnp.where(t >= 0.0, t, 0.2 * t)
    h = jnp.dot(a, w_ref[...], preferred_element_type=jnp.float32)
    o_ref[...] = h * dinv


def _tc_last(p_ref, hp_ref, degp_ref, b_ref, o_ref):
    dinv = _dinv_block(degp_ref[...])
    t = dinv * (p_ref[0] + p_ref[1] + hp_ref[...]) + b_ref[...]
    m = jnp.max(t, axis=1, keepdims=True)
    e = jnp.exp(t - m)
    lse = jnp.log(jnp.sum(e, axis=1, keepdims=True))
    o_ref[...] = (t - m) - lse


_row_spec = pl.BlockSpec((_BR, F), lambda i: (i, 0))
_part_spec = pl.BlockSpec((NC, _BR, F), lambda i: (0, i, 0))
_degp_spec = pl.BlockSpec((NC, _BR, 16), lambda i: (0, i, 0))
_w_spec = pl.BlockSpec((F, F), lambda i: (0, 0))
_b_spec = pl.BlockSpec((1, F), lambda i: (0, 0))
_grid = (NPAD // _BR,)
_out_rows = jax.ShapeDtypeStruct((NPAD, F), jnp.float32)


def kernel(x, edge_index, W1, b1, W2, b2):
    src = edge_index[0].astype(jnp.int32)
    dst = edge_index[1].astype(jnp.int32)
    e_per_w = N_EDGES // NW
    pad_per_w = SLOTS_PER_W - e_per_w
    padv = (N_NODES + (jnp.arange(NW * pad_per_w, dtype=jnp.int32)
                       % (NPAD - N_NODES))).reshape(NW, pad_per_w)
    srcf = jnp.concatenate([src.reshape(NW, e_per_w), padv], axis=1)
    dstf = jnp.concatenate([dst.reshape(NW, e_per_w), padv], axis=1)
    src3 = srcf.reshape(NW, STAGES, HCH, CHUNK)
    dst3 = dstf.reshape(NW, STAGES, HCH, CHUNK)
    dstd = dstf.reshape(NW, DCH, DCHUNK)

    xpad = jnp.zeros((NPAD, F), jnp.float32).at[:N_NODES].set(x)
    b1r = b1.reshape(1, F)
    b2r = b2.reshape(1, F)

    degp = _sc_degree(dstd)

    h1 = pl.pallas_call(
        _tc_matmul,
        grid=_grid,
        in_specs=[_row_spec, _w_spec],
        out_specs=_row_spec,
        out_shape=_out_rows,
    )(xpad, W1)

    h1p = pl.pallas_call(
        _tc_scale,
        grid=_grid,
        in_specs=[_row_spec, _degp_spec],
        out_specs=_row_spec,
        out_shape=_out_rows,
    )(h1, degp)

    p1 = _sc_propagate(h1p, src3, dst3)

    h2p = pl.pallas_call(
        _tc_mid,
        grid=_grid,
        in_specs=[_part_spec, _row_spec, _degp_spec, _b_spec, _w_spec],
        out_specs=_row_spec,
        out_shape=_out_rows,
    )(p1, h1p, degp, b1r, W2)

    p2 = _sc_propagate(h2p, src3, dst3)

    out = pl.pallas_call(
        _tc_last,
        grid=_grid,
        in_specs=[_part_spec, _row_spec, _degp_spec, _b_spec],
        out_specs=_row_spec,
        out_shape=_out_rows,
    )(p2, h2p, degp, b2r)

    return out[:N_NODES]
